# baseline (speedup 1.0000x reference)
import functools

import jax
import jax.numpy as jnp
from jax import lax
from jax.experimental import pallas as pl
from jax.experimental.pallas import tpu as pltpu
from jax.experimental.pallas import tpu_sc as plsc

NUM_CLASSES = 22
LANES = 16
NUM_CORES = 2
NUM_SUBCORES = 16
NUM_WORKERS = NUM_CORES * NUM_SUBCORES
SEQ = 1000000
PADSEQ = 1000064
CHUNK = 2048
N_FULL = SEQ // CHUNK
TAIL = SEQ - N_FULL * CHUNK
NBUF = 2
MAX_ITERS = (N_FULL + 1 + NUM_WORKERS - 1) // NUM_WORKERS
LAST = MAX_ITERS - 1
N_FULL_LAST = N_FULL - LAST * NUM_WORKERS
TAIL_WID = N_FULL_LAST


def _body(x_hbm, *rest):
    rows = rest[:NUM_CLASSES]
    (x0, x1, xt, b0, b1, bt, si0, si1, so0, so1) = rest[NUM_CLASSES:]
    wid = lax.axis_index("s") * NUM_CORES + lax.axis_index("c")

    x_bufs = [x0, x1]
    bufs = [b0, b1]
    in_sems = [si0, si1]
    out_sems = [so0, so1]

    def in_copy(i):
        o = (i * NUM_WORKERS + wid) * CHUNK
        return pltpu.make_async_copy(
            x_hbm.at[pl.ds(o, CHUNK)], x_bufs[i % NBUF], in_sems[i % NBUF]
        )

    class _Multi:
        def __init__(self, copies):
            self.copies = copies
        def start(self):
            for cp in self.copies:
                cp.start()
        def wait(self):
            for cp in self.copies:
                cp.wait()

    def out_copy(i):
        o = (i * NUM_WORKERS + wid) * CHUNK
        b = i % NBUF
        return _Multi([
            pltpu.make_async_copy(
                bufs[b].at[c], rows[c].at[pl.ds(o, CHUNK)], out_sems[b]
            )
            for c in range(NUM_CLASSES)
        ])

    def in_copy_tail():
        return pltpu.make_async_copy(
            x_hbm.at[pl.ds(N_FULL * CHUNK, TAIL)], xt, in_sems[LAST % NBUF]
        )

    def out_copy_tail():
        return _Multi([
            pltpu.make_async_copy(
                bt.at[c],
                rows[c].at[pl.ds(N_FULL * CHUNK, TAIL)],
                out_sems[LAST % NBUF],
            )
            for c in range(NUM_CLASSES)
        ])

    ones = jnp.full((LANES,), 1.0, jnp.float32)
    zeros = jnp.zeros((LANES,), jnp.float32)

    def compute(x_v, buf_v, width):
        def jbody(j, carry):
            xv = x_v[pl.ds(j * LANES, LANES)]
            for c in range(NUM_CLASSES):
                buf_v[c, pl.ds(j * LANES, LANES)] = jnp.where(
                    xv == c, ones, zeros
                )
            return carry

        lax.fori_loop(0, width // LANES, jbody, 0)

    def start_in(i):
        if i < LAST:
            in_copy(i).start()
        else:
            @pl.when(wid < N_FULL_LAST)
            def _():
                in_copy(i).start()

            @pl.when(wid == TAIL_WID)
            def _():
                in_copy_tail().start()

    for i in range(NBUF):
        start_in(i)

    for i in range(LAST):
        in_copy(i).wait()
        if i >= NBUF:
            out_copy(i - NBUF).wait()
        compute(x_bufs[i % NBUF], bufs[i % NBUF], CHUNK)
        out_copy(i).start()
        if i + NBUF < MAX_ITERS:
            start_in(i + NBUF)

    @pl.when(wid < N_FULL_LAST)
    def _():
        in_copy(LAST).wait()
        out_copy(LAST - NBUF).wait()
        compute(x_bufs[LAST % NBUF], bufs[LAST % NBUF], CHUNK)
        out_copy(LAST).start()

    @pl.when(wid == TAIL_WID)
    def _():
        in_copy_tail().wait()
        out_copy(LAST - NBUF).wait()
        compute(xt, bt, TAIL)
        out_copy_tail().start()

    out_copy(LAST - 1).wait()

    @pl.when(wid < N_FULL_LAST)
    def _():
        out_copy(LAST).wait()

    @pl.when(wid == TAIL_WID)
    def _():
        out_copy_tail().wait()


@jax.jit
def _onehot(x):
    fn = pl.kernel(
        _body,
        out_type=tuple(jax.ShapeDtypeStruct((SEQ,), jnp.float32) for _ in range(NUM_CLASSES)),
        mesh=plsc.VectorSubcoreMesh(core_axis_name="c", subcore_axis_name="s"),
        scratch_types=[
            pltpu.VMEM((CHUNK,), jnp.int32),
            pltpu.VMEM((CHUNK,), jnp.int32),
            pltpu.VMEM((TAIL,), jnp.int32),
            pltpu.VMEM((NUM_CLASSES, CHUNK), jnp.float32),
            pltpu.VMEM((NUM_CLASSES, CHUNK), jnp.float32),
            pltpu.VMEM((NUM_CLASSES, TAIL), jnp.float32),
            pltpu.SemaphoreType.DMA,
            pltpu.SemaphoreType.DMA,
            pltpu.SemaphoreType.DMA,
            pltpu.SemaphoreType.DMA,
        ],
        compiler_params=pltpu.CompilerParams(use_tc_tiling_on_sc=False),
    )
    return jnp.stack(fn(x), axis=0)


def kernel(x, table):
    del table
    return _onehot(x.astype(jnp.int32)).reshape(1, NUM_CLASSES, SEQ)


# speedup vs baseline: 2.5231x; 2.5231x over previous
import functools

import jax
import jax.numpy as jnp
from jax import lax
from jax.experimental import pallas as pl
from jax.experimental.pallas import tpu as pltpu
from jax.experimental.pallas import tpu_sc as plsc

NUM_CLASSES = 22
LANES = 16
NUM_CORES = 2
NUM_SUBCORES = 16
NUM_WORKERS = NUM_CORES * NUM_SUBCORES
SEQ = 1000000
PADSEQ = 1000064
CHUNK = 2048
N_FULL = SEQ // CHUNK
TAIL = SEQ - N_FULL * CHUNK
NBUF = 2
MAX_ITERS = (N_FULL + 1 + NUM_WORKERS - 1) // NUM_WORKERS
LAST = MAX_ITERS - 1
N_FULL_LAST = N_FULL - LAST * NUM_WORKERS
TAIL_WID = N_FULL_LAST


def _body(x_hbm, out_hbm, tail_hbm, x0, x1, xt, b0, b1, bt, si0, si1, so0, so1):
    wid = lax.axis_index("s") * NUM_CORES + lax.axis_index("c")

    x_bufs = [x0, x1]
    bufs = [b0, b1]
    in_sems = [si0, si1]
    out_sems = [so0, so1]

    def in_copy(i):
        o = (i * NUM_WORKERS + wid) * CHUNK
        return pltpu.make_async_copy(
            x_hbm.at[pl.ds(o, CHUNK)], x_bufs[i % NBUF], in_sems[i % NBUF]
        )

    def out_copy(i):
        o = (i * NUM_WORKERS + wid) * CHUNK
        return pltpu.make_async_copy(
            bufs[i % NBUF], out_hbm.at[:, pl.ds(o, CHUNK)], out_sems[i % NBUF]
        )

    def in_copy_tail():
        return pltpu.make_async_copy(
            x_hbm.at[pl.ds(N_FULL * CHUNK, TAIL)], xt, in_sems[LAST % NBUF]
        )

    def out_copy_tail():
        return pltpu.make_async_copy(bt, tail_hbm, out_sems[LAST % NBUF])

    ones = jnp.full((LANES,), 1.0, jnp.float32)
    zeros = jnp.zeros((LANES,), jnp.float32)

    def compute(x_v, buf_v, width):
        def jbody(j, carry):
            xv = x_v[pl.ds(j * LANES, LANES)]
            for c in range(NUM_CLASSES):
                buf_v[c, pl.ds(j * LANES, LANES)] = jnp.where(
                    xv == c, ones, zeros
                )
            return carry

        lax.fori_loop(0, width // LANES, jbody, 0)

    def start_in(i):
        if i < LAST:
            in_copy(i).start()
        else:
            @pl.when(wid < N_FULL_LAST)
            def _():
                in_copy(i).start()

            @pl.when(wid == TAIL_WID)
            def _():
                in_copy_tail().start()

    for i in range(NBUF):
        start_in(i)

    for i in range(LAST):
        in_copy(i).wait()
        if i >= NBUF:
            out_copy(i - NBUF).wait()
        compute(x_bufs[i % NBUF], bufs[i % NBUF], CHUNK)
        out_copy(i).start()
        if i + NBUF < MAX_ITERS:
            start_in(i + NBUF)

    @pl.when(wid < N_FULL_LAST)
    def _():
        in_copy(LAST).wait()
        out_copy(LAST - NBUF).wait()
        compute(x_bufs[LAST % NBUF], bufs[LAST % NBUF], CHUNK)
        out_copy(LAST).start()

    @pl.when(wid == TAIL_WID)
    def _():
        in_copy_tail().wait()
        out_copy(LAST - NBUF).wait()
        compute(xt, bt, TAIL)
        out_copy_tail().start()

    out_copy(LAST - 1).wait()

    @pl.when(wid < N_FULL_LAST)
    def _():
        out_copy(LAST).wait()

    @pl.when(wid == TAIL_WID)
    def _():
        out_copy_tail().wait()


@jax.jit
def _onehot(x):
    fn = pl.kernel(
        _body,
        out_type=(
            jax.ShapeDtypeStruct((NUM_CLASSES, SEQ), jnp.float32),
            jax.ShapeDtypeStruct((NUM_CLASSES, TAIL), jnp.float32),
        ),
        mesh=plsc.VectorSubcoreMesh(core_axis_name="c", subcore_axis_name="s"),
        scratch_types=[
            pltpu.VMEM((CHUNK,), jnp.int32),
            pltpu.VMEM((CHUNK,), jnp.int32),
            pltpu.VMEM((TAIL,), jnp.int32),
            pltpu.VMEM((NUM_CLASSES, CHUNK), jnp.float32),
            pltpu.VMEM((NUM_CLASSES, CHUNK), jnp.float32),
            pltpu.VMEM((NUM_CLASSES, TAIL), jnp.float32),
            pltpu.SemaphoreType.DMA,
            pltpu.SemaphoreType.DMA,
            pltpu.SemaphoreType.DMA,
            pltpu.SemaphoreType.DMA,
        ],
    )
    bulk, tail = fn(x)
    return lax.dynamic_update_slice(bulk, tail, (0, N_FULL * CHUNK))


def kernel(x, table):
    del table
    return _onehot(x.astype(jnp.int32)).reshape(1, NUM_CLASSES, SEQ)
